# Initial kernel scaffold; baseline (speedup 1.0000x reference)
import jax
import jax.numpy as jnp
from jax.experimental import pallas as pl


def kernel(vertex_features, edges, weights, W1, b1, W2, b2, W3, b3, W4, b4, Wl, bl):
    raise NotImplementedError("write your pallas kernel here")


# speedup vs baseline: 8.6209x; 8.6209x over previous
import functools

import jax
import jax.numpy as jnp
import numpy as np
from jax import lax
from jax.experimental import pallas as pl
from jax.experimental.pallas import tpu as pltpu
from jax.experimental.pallas import tpu_sc as plsc

NCORES = 2
NSUB = 16
LANES = 16
KPASS = 3
NRANGES = NCORES * KPASS


def _cdiv(a, b):
  return -(-a // b)


def _pick_stripe(n):
  tr = _cdiv(_cdiv(n, NRANGES * NSUB), 8) * 8
  while True:
    czs = [d for d in range(64, 257, 8) if tr % d == 0]
    if czs:
      return tr, max(czs)
    tr += 8


def _mesh():
  return plsc.VectorSubcoreMesh(
      core_axis_name="c", subcore_axis_name="s",
      num_cores=NCORES, num_subcores=NSUB)


def _deg_kernel(npad, ep, w):
  ept = ep // (NCORES * NSUB)
  nw = ept // w
  npt = npad // NSUB

  def body(col2d_hbm, ew_hbm, out_hbm, colb, ewb, zb, acc):
    c = lax.axis_index("c")
    s = lax.axis_index("s")
    base = (c * NSUB + s) * ept

    def zb_zero(i, _):
      zb[pl.ds(i * LANES, LANES)] = jnp.zeros((LANES,), jnp.float32)
      return 0
    lax.fori_loop(0, npt // LANES, zb_zero, 0)
    pltpu.sync_copy(zb, acc.at[pl.ds(pl.multiple_of(s * npt, npt), npt)])
    plsc.subcore_barrier()

    def window(wi, _):
      off = base + wi * w
      pltpu.sync_copy(
          col2d_hbm.at[pl.ds(pl.multiple_of(off // 128, w // 128), w // 128),
                       :], colb)
      pltpu.sync_copy(ew_hbm.at[pl.ds(pl.multiple_of(off, w), w)], ewb)

      def chunk(j, _):
        pltpu.sync_copy(ewb.at[pl.ds(j * 128, 128)], acc.at[colb.at[j]],
                        add=True)
        return 0
      lax.fori_loop(0, w // 128, chunk, 0)
      return 0
    lax.fori_loop(0, nw, window, 0)

    plsc.subcore_barrier()
    pltpu.sync_copy(acc.at[pl.ds(pl.multiple_of(s * npt, npt), npt)],
                    out_hbm.at[c, pl.ds(pl.multiple_of(s * npt, npt), npt)])

  return pl.kernel(
      body,
      out_type=jax.ShapeDtypeStruct((NCORES, npad), jnp.float32),
      mesh=_mesh(),
      scratch_types=[
          pltpu.VMEM((w // 128, 128), jnp.int32),
          pltpu.VMEM((w,), jnp.float32),
          pltpu.VMEM((npt,), jnp.float32),
          pltpu.VMEM_SHARED((npad,), jnp.float32),
      ],
      compiler_params=pltpu.CompilerParams(use_tc_tiling_on_sc=False,
                                           needs_layout_passes=False),
  )


def _spmm_kernel(n, npad, ep, w, rng, tr, cz, h):
  ept = ep // NSUB
  nw = ept // w
  gb = 128

  def body(row_hbm, col_hbm, ew_hbm, u_hbm, out_hbm,
           rowb, colb, ewb, crow, ccol, cew, gbuf, sidx, zb, acc, sem):
    c = lax.axis_index("c")
    s = lax.axis_index("s")

    def zb_zero(i, _):
      zb[i, pl.ds(0, LANES)] = jnp.zeros((LANES,), jnp.float32)
      zb[i, pl.ds(LANES, LANES)] = jnp.zeros((LANES,), jnp.float32)
      zb[i, pl.ds(2 * LANES, LANES)] = jnp.zeros((LANES,), jnp.float32)
      zb[i, pl.ds(3 * LANES, LANES)] = jnp.zeros((LANES,), jnp.float32)
      return 0
    lax.fori_loop(0, cz, zb_zero, 0)

    iota16 = lax.iota(jnp.int32, LANES)

    for p in range(KPASS):
      rix = c * KPASS + p
      lo = rix * rng

      for z in range(tr // cz):
        pltpu.sync_copy(
            zb, acc.at[pl.ds(pl.multiple_of(s * tr + z * cz, cz), cz), :])
      plsc.subcore_barrier()

      def window(wi, _):
        off = pl.multiple_of(s * ept + wi * w, w)
        pltpu.sync_copy(row_hbm.at[pl.ds(off, w)], rowb)
        pltpu.sync_copy(col_hbm.at[pl.ds(off, w)], colb)
        pltpu.sync_copy(ew_hbm.at[pl.ds(off, w)], ewb)

        def scan(i, m):
          cols = colb[pl.ds(i * LANES, LANES)]
          msk = (cols >= lo) & (cols < lo + rng)
          p = msk.astype(jnp.int32)
          for sh in (1, 2, 4, 8):
            idx = jnp.maximum(iota16 - sh, 0)
            g = p.at[idx].get(mode="promise_in_bounds")
            p = p + jnp.where(iota16 >= sh, g, 0)
          pos = m + p - 1
          plsc.store_scatter(crow, [pos],
                             rowb[pl.ds(i * LANES, LANES)], mask=msk)
          plsc.store_scatter(ccol, [pos], cols - lo, mask=msk)
          plsc.store_scatter(cew, [pos],
                             ewb[pl.ds(i * LANES, LANES)], mask=msk)
          return m + jnp.squeeze(lax.slice(p, (LANES - 1,), (LANES,)))
        m = lax.fori_loop(0, w // LANES, scan, 0)

        for i in range(gb // LANES):
          crow[pl.ds(m + i * LANES, LANES)] = iota16
          ccol[pl.ds(m + i * LANES, LANES)] = iota16
          cew[pl.ds(m + i * LANES, LANES)] = jnp.zeros((LANES,), jnp.float32)

        nb = (m + gb - 1) // gb

        def chunk(g, _):
          base = g * gb
          pltpu.async_copy(u_hbm.at[crow.at[pl.ds(base, gb)]], gbuf,
                           sem).wait()

          def mul(q, _):
            ewv = cew[pl.ds(base + q * LANES, LANES)]
            for e in range(LANES):
              sc = ewv.at[jnp.full((LANES,), e, jnp.int32)].get(
                  mode="promise_in_bounds")
              r = q * LANES + e
              for fb in range(h // LANES):
                cur = gbuf[r, pl.ds(fb * LANES, LANES)]
                gbuf[r, pl.ds(fb * LANES, LANES)] = cur * sc
            return 0
          lax.fori_loop(0, gb // LANES, mul, 0)

          for kk in range(gb // LANES):
            sidx[0, pl.ds(kk * LANES, LANES)] = (
                ccol[pl.ds(base + kk * LANES, LANES)])
          pltpu.sync_copy(gbuf, acc.at[sidx.at[0]], add=True)
          return 0
        lax.fori_loop(0, nb, chunk, 0)
        return 0
      lax.fori_loop(0, nw, window, 0)

      plsc.subcore_barrier()
      for z in range(tr // cz):
        stripe = pl.multiple_of(s * tr + z * cz, cz)
        dsto = pl.multiple_of(lo + s * tr + z * cz, cz)
        pltpu.sync_copy(acc.at[pl.ds(stripe, cz), :],
                        out_hbm.at[pl.ds(dsto, cz), :])
      plsc.subcore_barrier()

  return pl.kernel(
      body,
      out_type=jax.ShapeDtypeStruct((npad, h), jnp.float32),
      mesh=_mesh(),
      scratch_types=[
          pltpu.VMEM((w,), jnp.int32),
          pltpu.VMEM((w,), jnp.int32),
          pltpu.VMEM((w,), jnp.float32),
          pltpu.VMEM((w + 128,), jnp.int32),
          pltpu.VMEM((w + 128,), jnp.int32),
          pltpu.VMEM((w + 128,), jnp.float32),
          pltpu.VMEM((128, h), jnp.float32),
          pltpu.VMEM((1, 128), jnp.int32),
          pltpu.VMEM((cz, h), jnp.float32),
          pltpu.VMEM_SHARED((rng, h), jnp.float32),
          pltpu.SemaphoreType.DMA,
      ],
      compiler_params=pltpu.CompilerParams(use_tc_tiling_on_sc=False, needs_layout_passes=False),
  )


def _tc_first(x, w1, degsum, bn):
  n, din = x.shape
  h = w1.shape[1]
  grid = (n // bn,)

  def body(xb, wb, db, xw_o, u_o, dis_o):
    d = lax.rsqrt(db[...])
    xw = jnp.dot(xb[...], wb[...], preferred_element_type=jnp.float32)
    xw_o[...] = xw
    u_o[...] = xw * d
    dis_o[...] = d

  return pl.pallas_call(
      body,
      grid=grid,
      in_specs=[
          pl.BlockSpec((bn, din), lambda i: (i, 0)),
          pl.BlockSpec((din, h), lambda i: (0, 0)),
          pl.BlockSpec((bn, 1), lambda i: (i, 0)),
      ],
      out_specs=[
          pl.BlockSpec((bn, h), lambda i: (i, 0)),
          pl.BlockSpec((bn, h), lambda i: (i, 0)),
          pl.BlockSpec((bn, 1), lambda i: (i, 0)),
      ],
      out_shape=[
          jax.ShapeDtypeStruct((n, h), jnp.float32),
          jax.ShapeDtypeStruct((n, h), jnp.float32),
          jax.ShapeDtypeStruct((n, 1), jnp.float32),
      ],
  )(x, w1, degsum)


def _tc_mid(s_in, xw, dis, b, wn, bn):
  n, h = xw.shape
  grid = (n // bn,)

  def body(sb, xwb, db, bb, wb, xwn_o, un_o):
    d = db[...]
    pre = d * sb[...] + (d * d) * xwb[...] + bb[...]
    hact = jax.nn.sigmoid(pre)
    xwn = jnp.dot(hact, wb[...], preferred_element_type=jnp.float32)
    xwn_o[...] = xwn
    un_o[...] = xwn * d

  return pl.pallas_call(
      body,
      grid=grid,
      in_specs=[
          pl.BlockSpec((bn, h), lambda i: (i, 0)),
          pl.BlockSpec((bn, h), lambda i: (i, 0)),
          pl.BlockSpec((bn, 1), lambda i: (i, 0)),
          pl.BlockSpec((1, h), lambda i: (0, 0)),
          pl.BlockSpec((h, h), lambda i: (0, 0)),
      ],
      out_specs=[
          pl.BlockSpec((bn, h), lambda i: (i, 0)),
          pl.BlockSpec((bn, h), lambda i: (i, 0)),
      ],
      out_shape=[
          jax.ShapeDtypeStruct((n, h), jnp.float32),
          jax.ShapeDtypeStruct((n, h), jnp.float32),
      ],
  )(s_in, xw, dis, b, wn)


def _tc_final(s_in, xw, dis, b, wl, bl):
  m, h = xw.shape
  out = wl.shape[1]

  def body(sb, xwb, db, bb, wb, blb, o):
    d = db[...]
    pre = d * sb[...] + (d * d) * xwb[...] + bb[...]
    hact = jax.nn.sigmoid(pre)
    o[...] = jnp.dot(hact, wb[...],
                     preferred_element_type=jnp.float32) + blb[...]

  return pl.pallas_call(
      body,
      out_shape=jax.ShapeDtypeStruct((m, out), jnp.float32),
  )(s_in, xw, dis, b, wl, bl)


def kernel(vertex_features, edges, weights, W1, b1, W2, b2, W3, b3, W4, b4,
           Wl, bl):
  n, din = vertex_features.shape
  e = edges.shape[1]
  h = W1.shape[1]

  tr, cz = _pick_stripe(n)
  npad = NRANGES * NSUB * tr
  rng = npad // NRANGES

  w = 4096
  ept = _cdiv(e, NSUB * w) * w
  ep = ept * NSUB
  pad = ep - e
  row = edges[0]
  col = edges[1]
  ew = weights
  if pad:
    pidx = lax.rem(lax.iota(jnp.int32, pad), jnp.int32(n))
    row = jnp.concatenate([row, pidx])
    col = jnp.concatenate([col, pidx])
    ew = jnp.concatenate([ew, jnp.zeros((pad,), jnp.float32)])
  col2d = col.reshape(ep // 128, 128)

  deg_part = _deg_kernel(npad, ep, w // NCORES)(col2d, ew)
  degsum = (deg_part[0] + deg_part[1] + 1.0)[:n].reshape(n, 1)

  bn = 2000 if n % 2000 == 0 else 8 * (n // 8)
  while n % bn:
    bn -= 8

  xw, u, dis = _tc_first(vertex_features, W1, degsum, bn)

  spmm = _spmm_kernel(n, npad, ep, w, rng, tr, cz, h)
  for wnext, bcur in ((W2, b1), (W3, b2), (W4, b3)):
    s_out = spmm(row, col, ew, u)
    xw, u = _tc_mid(s_out[:n], xw, dis, bcur.reshape(1, h), wnext, bn)

  s_out = spmm(row, col, ew, u)
  head = 32
  emb = _tc_final(s_out[:head], xw[:head], dis[:head],
                  b4.reshape(1, h), Wl, bl.reshape(1, Wl.shape[1]))
  return emb[:19]


# speedup vs baseline: 10.8033x; 1.2532x over previous
import functools

import jax
import jax.numpy as jnp
import numpy as np
from jax import lax
from jax.experimental import pallas as pl
from jax.experimental.pallas import tpu as pltpu
from jax.experimental.pallas import tpu_sc as plsc

NCORES = 2
NSUB = 16
LANES = 16
KPASS = 3
NRANGES = NCORES * KPASS


def _cdiv(a, b):
  return -(-a // b)


def _pick_stripe(n):
  tr = _cdiv(_cdiv(n, NRANGES * NSUB), 8) * 8
  while True:
    czs = [d for d in range(64, 257, 8) if tr % d == 0]
    if czs:
      return tr, max(czs)
    tr += 8


def _mesh():
  return plsc.VectorSubcoreMesh(
      core_axis_name="c", subcore_axis_name="s",
      num_cores=NCORES, num_subcores=NSUB)


def _deg_kernel(npad, ep, w):
  ept = ep // (NCORES * NSUB)
  nw = ept // w
  npt = npad // NSUB

  def body(col2d_hbm, ew_hbm, out_hbm, colb, ewb, zb, acc):
    c = lax.axis_index("c")
    s = lax.axis_index("s")
    base = (c * NSUB + s) * ept

    def zb_zero(i, _):
      zb[pl.ds(i * LANES, LANES)] = jnp.zeros((LANES,), jnp.float32)
      return 0
    lax.fori_loop(0, npt // LANES, zb_zero, 0)
    pltpu.sync_copy(zb, acc.at[pl.ds(pl.multiple_of(s * npt, npt), npt)])
    plsc.subcore_barrier()

    def window(wi, _):
      off = base + wi * w
      pltpu.sync_copy(
          col2d_hbm.at[pl.ds(pl.multiple_of(off // 128, w // 128), w // 128),
                       :], colb)
      pltpu.sync_copy(ew_hbm.at[pl.ds(pl.multiple_of(off, w), w)], ewb)

      def chunk(j, _):
        pltpu.sync_copy(ewb.at[pl.ds(j * 128, 128)], acc.at[colb.at[j]],
                        add=True)
        return 0
      lax.fori_loop(0, w // 128, chunk, 0)
      return 0
    lax.fori_loop(0, nw, window, 0)

    plsc.subcore_barrier()
    pltpu.sync_copy(acc.at[pl.ds(pl.multiple_of(s * npt, npt), npt)],
                    out_hbm.at[c, pl.ds(pl.multiple_of(s * npt, npt), npt)])

  return pl.kernel(
      body,
      out_type=jax.ShapeDtypeStruct((NCORES, npad), jnp.float32),
      mesh=_mesh(),
      scratch_types=[
          pltpu.VMEM((w // 128, 128), jnp.int32),
          pltpu.VMEM((w,), jnp.float32),
          pltpu.VMEM((npt,), jnp.float32),
          pltpu.VMEM_SHARED((npad,), jnp.float32),
      ],
      compiler_params=pltpu.CompilerParams(use_tc_tiling_on_sc=False,
                                           needs_layout_passes=False),
  )


def _spmm_kernel(n, npad, ep, w, rng, tr, cz, h):
  ept = ep // NSUB
  nw = ept // w
  gb = 128

  def body(row_hbm, col_hbm, ew_hbm, u_hbm, out_hbm,
           rowb, colb, ewb, crow, ccol, cew, gbuf, sidx, zb, acc, sem):
    c = lax.axis_index("c")
    s = lax.axis_index("s")

    def zb_zero(i, _):
      zb[i, pl.ds(0, LANES)] = jnp.zeros((LANES,), jnp.float32)
      zb[i, pl.ds(LANES, LANES)] = jnp.zeros((LANES,), jnp.float32)
      zb[i, pl.ds(2 * LANES, LANES)] = jnp.zeros((LANES,), jnp.float32)
      zb[i, pl.ds(3 * LANES, LANES)] = jnp.zeros((LANES,), jnp.float32)
      return 0
    lax.fori_loop(0, cz, zb_zero, 0)

    iota16 = lax.iota(jnp.int32, LANES)

    for p in range(KPASS):
      rix = c * KPASS + p
      lo = rix * rng

      for z in range(tr // cz):
        pltpu.sync_copy(
            zb, acc.at[pl.ds(pl.multiple_of(s * tr + z * cz, cz), cz), :])
      plsc.subcore_barrier()

      def window(wi, _):
        off = pl.multiple_of(s * ept + wi * w, w)
        pltpu.sync_copy(row_hbm.at[pl.ds(off, w)], rowb)
        pltpu.sync_copy(col_hbm.at[pl.ds(off, w)], colb)
        pltpu.sync_copy(ew_hbm.at[pl.ds(off, w)], ewb)

        def scan(i, m):
          cols = colb[pl.ds(i * LANES, LANES)]
          msk = (cols >= lo) & (cols < lo + rng)
          p = msk.astype(jnp.int32)
          for sh in (1, 2, 4, 8):
            idx = jnp.maximum(iota16 - sh, 0)
            g = p.at[idx].get(mode="promise_in_bounds")
            p = p + jnp.where(iota16 >= sh, g, 0)
          pos = m + p - 1
          plsc.store_scatter(crow, [pos],
                             rowb[pl.ds(i * LANES, LANES)], mask=msk)
          plsc.store_scatter(ccol, [pos], cols - lo, mask=msk)
          plsc.store_scatter(cew, [pos],
                             ewb[pl.ds(i * LANES, LANES)], mask=msk)
          return m + jnp.squeeze(lax.slice(p, (LANES - 1,), (LANES,)))
        m = lax.fori_loop(0, w // LANES, scan, 0)

        for i in range(gb // LANES):
          crow[pl.ds(m + i * LANES, LANES)] = iota16
          ccol[pl.ds(m + i * LANES, LANES)] = iota16
          cew[pl.ds(m + i * LANES, LANES)] = jnp.zeros((LANES,), jnp.float32)

        nb = (m + gb - 1) // gb

        def issue(g, slot):
          pltpu.async_copy(
              u_hbm.at[crow.at[pl.ds(g * gb, gb)]],
              gbuf.at[pl.ds(slot * gb, gb), :], sem.at[slot])

        @pl.when(nb > 0)
        def _():
          issue(0, 0)

        def chunk(g, _):
          slot = lax.rem(g, 2)
          nslot = 1 - slot

          @pl.when(g + 1 < nb)
          def _():
            issue(g + 1, nslot)

          pltpu.make_async_copy(
              u_hbm.at[crow.at[pl.ds(g * gb, gb)]],
              gbuf.at[pl.ds(slot * gb, gb), :], sem.at[slot]).wait()

          rbase = slot * gb
          base = g * gb

          def mul(q, _):
            ewv = cew[pl.ds(base + q * LANES, LANES)]
            for e in range(LANES):
              sc = ewv.at[jnp.full((LANES,), e, jnp.int32)].get(
                  mode="promise_in_bounds")
              r = rbase + q * LANES + e
              for fb in range(h // LANES):
                cur = gbuf[r, pl.ds(fb * LANES, LANES)]
                gbuf[r, pl.ds(fb * LANES, LANES)] = cur * sc
            return 0
          lax.fori_loop(0, gb // LANES, mul, 0)

          for kk in range(gb // LANES):
            sidx[0, pl.ds(kk * LANES, LANES)] = (
                ccol[pl.ds(base + kk * LANES, LANES)])
          pltpu.sync_copy(gbuf.at[pl.ds(rbase, gb), :], acc.at[sidx.at[0]],
                          add=True)
          return 0
        lax.fori_loop(0, nb, chunk, 0)
        return 0
      lax.fori_loop(0, nw, window, 0)

      plsc.subcore_barrier()
      for z in range(tr // cz):
        stripe = pl.multiple_of(s * tr + z * cz, cz)
        dsto = pl.multiple_of(lo + s * tr + z * cz, cz)
        pltpu.sync_copy(acc.at[pl.ds(stripe, cz), :],
                        out_hbm.at[pl.ds(dsto, cz), :])
      plsc.subcore_barrier()

  return pl.kernel(
      body,
      out_type=jax.ShapeDtypeStruct((npad, h), jnp.float32),
      mesh=_mesh(),
      scratch_types=[
          pltpu.VMEM((w,), jnp.int32),
          pltpu.VMEM((w,), jnp.int32),
          pltpu.VMEM((w,), jnp.float32),
          pltpu.VMEM((w + 128,), jnp.int32),
          pltpu.VMEM((w + 128,), jnp.int32),
          pltpu.VMEM((w + 128,), jnp.float32),
          pltpu.VMEM((2 * 128, h), jnp.float32),
          pltpu.VMEM((1, 128), jnp.int32),
          pltpu.VMEM((cz, h), jnp.float32),
          pltpu.VMEM_SHARED((rng, h), jnp.float32),
          pltpu.SemaphoreType.DMA((2,)),
      ],
      compiler_params=pltpu.CompilerParams(use_tc_tiling_on_sc=False, needs_layout_passes=False),
  )


def _tc_first(x, w1, degsum, bn):
  n, din = x.shape
  h = w1.shape[1]
  grid = (n // bn,)

  def body(xb, wb, db, xw_o, u_o, dis_o):
    d = lax.rsqrt(db[...])
    xw = jnp.dot(xb[...], wb[...], preferred_element_type=jnp.float32)
    xw_o[...] = xw
    u_o[...] = xw * d
    dis_o[...] = d

  return pl.pallas_call(
      body,
      grid=grid,
      in_specs=[
          pl.BlockSpec((bn, din), lambda i: (i, 0)),
          pl.BlockSpec((din, h), lambda i: (0, 0)),
          pl.BlockSpec((bn, 1), lambda i: (i, 0)),
      ],
      out_specs=[
          pl.BlockSpec((bn, h), lambda i: (i, 0)),
          pl.BlockSpec((bn, h), lambda i: (i, 0)),
          pl.BlockSpec((bn, 1), lambda i: (i, 0)),
      ],
      out_shape=[
          jax.ShapeDtypeStruct((n, h), jnp.float32),
          jax.ShapeDtypeStruct((n, h), jnp.float32),
          jax.ShapeDtypeStruct((n, 1), jnp.float32),
      ],
  )(x, w1, degsum)


def _tc_mid(s_in, xw, dis, b, wn, bn):
  n, h = xw.shape
  grid = (n // bn,)

  def body(sb, xwb, db, bb, wb, xwn_o, un_o):
    d = db[...]
    pre = d * sb[...] + (d * d) * xwb[...] + bb[...]
    hact = jax.nn.sigmoid(pre)
    xwn = jnp.dot(hact, wb[...], preferred_element_type=jnp.float32)
    xwn_o[...] = xwn
    un_o[...] = xwn * d

  return pl.pallas_call(
      body,
      grid=grid,
      in_specs=[
          pl.BlockSpec((bn, h), lambda i: (i, 0)),
          pl.BlockSpec((bn, h), lambda i: (i, 0)),
          pl.BlockSpec((bn, 1), lambda i: (i, 0)),
          pl.BlockSpec((1, h), lambda i: (0, 0)),
          pl.BlockSpec((h, h), lambda i: (0, 0)),
      ],
      out_specs=[
          pl.BlockSpec((bn, h), lambda i: (i, 0)),
          pl.BlockSpec((bn, h), lambda i: (i, 0)),
      ],
      out_shape=[
          jax.ShapeDtypeStruct((n, h), jnp.float32),
          jax.ShapeDtypeStruct((n, h), jnp.float32),
      ],
  )(s_in, xw, dis, b, wn)


def _tc_final(s_in, xw, dis, b, wl, bl):
  m, h = xw.shape
  out = wl.shape[1]

  def body(sb, xwb, db, bb, wb, blb, o):
    d = db[...]
    pre = d * sb[...] + (d * d) * xwb[...] + bb[...]
    hact = jax.nn.sigmoid(pre)
    o[...] = jnp.dot(hact, wb[...],
                     preferred_element_type=jnp.float32) + blb[...]

  return pl.pallas_call(
      body,
      out_shape=jax.ShapeDtypeStruct((m, out), jnp.float32),
  )(s_in, xw, dis, b, wl, bl)


def kernel(vertex_features, edges, weights, W1, b1, W2, b2, W3, b3, W4, b4,
           Wl, bl):
  n, din = vertex_features.shape
  e = edges.shape[1]
  h = W1.shape[1]

  tr, cz = _pick_stripe(n)
  npad = NRANGES * NSUB * tr
  rng = npad // NRANGES

  w = 4096
  ept = _cdiv(e, NSUB * w) * w
  ep = ept * NSUB
  pad = ep - e
  row = edges[0]
  col = edges[1]
  ew = weights
  if pad:
    pidx = lax.rem(lax.iota(jnp.int32, pad), jnp.int32(n))
    row = jnp.concatenate([row, pidx])
    col = jnp.concatenate([col, pidx])
    ew = jnp.concatenate([ew, jnp.zeros((pad,), jnp.float32)])
  col2d = col.reshape(ep // 128, 128)

  deg_part = _deg_kernel(npad, ep, w // NCORES)(col2d, ew)
  degsum = (deg_part[0] + deg_part[1] + 1.0)[:n].reshape(n, 1)

  bn = 2000 if n % 2000 == 0 else 8 * (n // 8)
  while n % bn:
    bn -= 8

  xw, u, dis = _tc_first(vertex_features, W1, degsum, bn)

  spmm = _spmm_kernel(n, npad, ep, w, rng, tr, cz, h)
  for wnext, bcur in ((W2, b1), (W3, b2), (W4, b3)):
    s_out = spmm(row, col, ew, u)
    xw, u = _tc_mid(s_out[:n], xw, dis, bcur.reshape(1, h), wnext, bn)

  s_out = spmm(row, col, ew, u)
  head = 32
  emb = _tc_final(s_out[:head], xw[:head], dis[:head],
                  b4.reshape(1, h), Wl, bl.reshape(1, Wl.shape[1]))
  return emb[:19]


# speedup vs baseline: 11.0914x; 1.0267x over previous
import jax
import jax.numpy as jnp
from jax import lax
from jax.experimental import pallas as pl
from jax.experimental.pallas import tpu as pltpu
from jax.experimental.pallas import tpu_sc as plsc

NCORES = 2
NSUB = 16
LANES = 16
KPASS = 3
NRANGES = NCORES * KPASS
NW = NCORES * NSUB
GB = 128
WP = 2048

_SC_PARAMS = pltpu.CompilerParams(use_tc_tiling_on_sc=False,
                                  needs_layout_passes=False)


def _cdiv(a, b):
  return -(-a // b)


def _pick_stripe(n):
  tr = _cdiv(_cdiv(n, NRANGES * NSUB), 8) * 8
  while True:
    czs = [d for d in range(64, 257, 8) if tr % d == 0]
    if czs:
      return tr, max(czs)
    tr += 8


def _mesh():
  return plsc.VectorSubcoreMesh(
      core_axis_name="c", subcore_axis_name="s",
      num_cores=NCORES, num_subcores=NSUB)


def _part_kernel(npad, ep, rng, capb):
  capw = ep // NW
  nwp = capw // WP
  npt = npad // NSUB
  pb = WP + GB

  def body(row_hbm, col_hbm, ew_hbm,
           prow_o, pcol_o, pew_o, cnt_o, deg_o,
           rowb, colb, ewb, col2db, pbrow, pbcol, pbew, zb, cbuf, acc,
           sem_f, sem_d):
    c = lax.axis_index("c")
    s = lax.axis_index("s")
    wid = c * NSUB + s
    base = wid * capw
    iota16 = lax.iota(jnp.int32, LANES)
    zeros16f = jnp.zeros((LANES,), jnp.float32)

    def zb_zero(i, _):
      zb[pl.ds(i * LANES, LANES)] = zeros16f
      return 0
    lax.fori_loop(0, npt // LANES, zb_zero, 0)
    pltpu.sync_copy(zb, acc.at[pl.ds(pl.multiple_of(s * npt, npt), npt)])
    plsc.subcore_barrier()

    def window(wi, carry):
      off = pl.multiple_of(base + wi * WP, WP)
      pltpu.sync_copy(row_hbm.at[pl.ds(off, WP)], rowb)
      pltpu.sync_copy(col_hbm.at[pl.ds(off, WP)], colb)
      pltpu.sync_copy(ew_hbm.at[pl.ds(off, WP)], ewb)

      def dchunk(j, _):
        for k in range(128 // LANES):
          col2db[j, pl.ds(k * LANES, LANES)] = (
              colb[pl.ds(j * 128 + k * LANES, LANES)])
        pltpu.async_copy(ewb.at[pl.ds(j * 128, 128)], acc.at[col2db.at[j]],
                         sem_d, add=True)
        return 0
      lax.fori_loop(0, WP // 128, dchunk, 0)

      tails = []
      offbs = []
      nfs = []
      for r in range(NRANGES):
        lo = r * rng

        def scan(i, t):
          cols = colb[pl.ds(i * LANES, LANES)]
          msk = (cols >= lo) & (cols < lo + rng)
          mi = msk.astype(jnp.int32)
          pos = t + plsc.cumsum(mi) - 1
          plsc.store_scatter(pbrow.at[r], [pos],
                             rowb[pl.ds(i * LANES, LANES)], mask=msk)
          plsc.store_scatter(pbcol.at[r], [pos], cols - lo, mask=msk)
          plsc.store_scatter(pbew.at[r], [pos],
                             ewb[pl.ds(i * LANES, LANES)], mask=msk)
          return t + jnp.sum(mi)
        t2 = lax.fori_loop(0, WP // LANES, scan, carry[r])
        nf = t2 // GB
        offb = carry[NRANGES + r]

        def flush(b, _):
          src = pl.multiple_of(b * GB, GB)
          dst = pl.multiple_of((offb + b) * GB, GB)
          pltpu.async_copy(pbrow.at[r, pl.ds(src, GB)],
                           prow_o.at[r, wid, pl.ds(dst, GB)], sem_f)
          pltpu.async_copy(pbcol.at[r, pl.ds(src, GB)],
                           pcol_o.at[r, wid, pl.ds(dst, GB)], sem_f)
          pltpu.async_copy(pbew.at[r, pl.ds(src, GB)],
                           pew_o.at[r, wid, pl.ds(dst, GB)], sem_f)
          return 0
        lax.fori_loop(0, nf, flush, 0)
        tails.append(t2 - nf * GB)
        offbs.append(offb + nf)
        nfs.append(nf)

      for r in range(NRANGES):
        offb = carry[NRANGES + r]

        def fwait(b, _):
          src = pl.multiple_of(b * GB, GB)
          dst = pl.multiple_of((offb + b) * GB, GB)
          pltpu.make_async_copy(pbrow.at[r, pl.ds(src, GB)],
                                prow_o.at[r, wid, pl.ds(dst, GB)],
                                sem_f).wait()
          pltpu.make_async_copy(pbcol.at[r, pl.ds(src, GB)],
                                pcol_o.at[r, wid, pl.ds(dst, GB)],
                                sem_f).wait()
          pltpu.make_async_copy(pbew.at[r, pl.ds(src, GB)],
                                pew_o.at[r, wid, pl.ds(dst, GB)],
                                sem_f).wait()
          return 0
        lax.fori_loop(0, nfs[r], fwait, 0)

      pltpu.make_async_copy(ew_hbm.at[pl.ds(off, WP)], ewb, sem_d).wait()

      for r in range(NRANGES):
        srcoff = pl.multiple_of(nfs[r] * GB, GB)
        for k in range(GB // LANES):
          vr = pbrow[r, pl.ds(srcoff + k * LANES, LANES)]
          vc = pbcol[r, pl.ds(srcoff + k * LANES, LANES)]
          ve = pbew[r, pl.ds(srcoff + k * LANES, LANES)]
          pbrow[r, pl.ds(k * LANES, LANES)] = vr
          pbcol[r, pl.ds(k * LANES, LANES)] = vc
          pbew[r, pl.ds(k * LANES, LANES)] = ve

      return tuple(tails) + tuple(offbs)

    carry = lax.fori_loop(0, nwp, window,
                          tuple(jnp.int32(0) for _ in range(2 * NRANGES)))

    cv = jnp.zeros((LANES,), jnp.int32)
    for r in range(NRANGES):
      tail = carry[r]
      offb = carry[NRANGES + r]
      for i in range(GB // LANES):
        pos = tail + i * LANES + iota16
        plsc.store_scatter(pbrow.at[r], [pos], iota16)
        plsc.store_scatter(pbcol.at[r], [pos], iota16)
        plsc.store_scatter(pbew.at[r], [pos], zeros16f)
      dst = pl.multiple_of(offb * GB, GB)
      pltpu.async_copy(pbrow.at[r, pl.ds(0, GB)],
                       prow_o.at[r, wid, pl.ds(dst, GB)], sem_f)
      pltpu.async_copy(pbcol.at[r, pl.ds(0, GB)],
                       pcol_o.at[r, wid, pl.ds(dst, GB)], sem_f)
      pltpu.async_copy(pbew.at[r, pl.ds(0, GB)],
                       pew_o.at[r, wid, pl.ds(dst, GB)], sem_f)
      cv = cv + jnp.where(iota16 == r, offb + 1, 0)
    for r in range(NRANGES):
      offb = carry[NRANGES + r]
      dst = pl.multiple_of(offb * GB, GB)
      pltpu.make_async_copy(pbrow.at[r, pl.ds(0, GB)],
                            prow_o.at[r, wid, pl.ds(dst, GB)], sem_f).wait()
      pltpu.make_async_copy(pbcol.at[r, pl.ds(0, GB)],
                            pcol_o.at[r, wid, pl.ds(dst, GB)], sem_f).wait()
      pltpu.make_async_copy(pbew.at[r, pl.ds(0, GB)],
                            pew_o.at[r, wid, pl.ds(dst, GB)], sem_f).wait()

    cbuf[pl.ds(0, LANES)] = cv
    pltpu.sync_copy(cbuf, cnt_o.at[wid])

    plsc.subcore_barrier()
    pltpu.sync_copy(acc.at[pl.ds(pl.multiple_of(s * npt, npt), npt)],
                    deg_o.at[c, pl.ds(pl.multiple_of(s * npt, npt), npt)])

  return pl.kernel(
      body,
      out_type=[
          jax.ShapeDtypeStruct((NRANGES, NW, capb), jnp.int32),
          jax.ShapeDtypeStruct((NRANGES, NW, capb), jnp.int32),
          jax.ShapeDtypeStruct((NRANGES, NW, capb), jnp.float32),
          jax.ShapeDtypeStruct((NW, LANES), jnp.int32),
          jax.ShapeDtypeStruct((NCORES, npad), jnp.float32),
      ],
      mesh=_mesh(),
      scratch_types=[
          pltpu.VMEM((WP,), jnp.int32),
          pltpu.VMEM((WP,), jnp.int32),
          pltpu.VMEM((WP,), jnp.float32),
          pltpu.VMEM((WP // 128, 128), jnp.int32),
          pltpu.VMEM((NRANGES, pb), jnp.int32),
          pltpu.VMEM((NRANGES, pb), jnp.int32),
          pltpu.VMEM((NRANGES, pb), jnp.float32),
          pltpu.VMEM((npad // NSUB,), jnp.float32),
          pltpu.VMEM((LANES,), jnp.int32),
          pltpu.VMEM_SHARED((npad,), jnp.float32),
          pltpu.SemaphoreType.DMA,
          pltpu.SemaphoreType.DMA,
      ],
      compiler_params=_SC_PARAMS,
  )


def _spmm_kernel(npad, rng, tr, cz, h, capb):
  def body(prow_hbm, pcol_hbm, pew_hbm, cnt_hbm, u_hbm, out_hbm,
           idxb, sidx, ewst, gbuf, zb, cntb, acc, sem_st, sem_g):
    c = lax.axis_index("c")
    s = lax.axis_index("s")

    def zb_zero(i, _):
      for fb in range(h // LANES):
        zb[i, pl.ds(fb * LANES, LANES)] = jnp.zeros((LANES,), jnp.float32)
      return 0
    lax.fori_loop(0, cz, zb_zero, 0)

    pltpu.sync_copy(cnt_hbm, cntb)

    for p in range(KPASS):
      rix = c * KPASS + p
      lo = rix * rng

      for z in range(tr // cz):
        pltpu.sync_copy(
            zb, acc.at[pl.ds(pl.multiple_of(s * tr + z * cz, cz), cz), :])
      plsc.subcore_barrier()

      for wsub in range(2):
        wid2 = s * 2 + wsub
        cvec = cntb[wid2, pl.ds(0, LANES)]
        nbv = cvec.at[jnp.full((LANES,), rix, jnp.int32)].get(
            mode="promise_in_bounds")
        nb = jnp.squeeze(lax.slice(nbv, (0,), (1,)))

        def stage(g, t3):
          o = pl.multiple_of(g * GB, GB)
          pltpu.async_copy(prow_hbm.at[rix, wid2, pl.ds(o, GB)],
                           idxb.at[t3], sem_st.at[t3])
          pltpu.async_copy(pcol_hbm.at[rix, wid2, pl.ds(o, GB)],
                           sidx.at[t3], sem_st.at[t3])
          pltpu.async_copy(pew_hbm.at[rix, wid2, pl.ds(o, GB)],
                           ewst.at[t3], sem_st.at[t3])

        def stage_wait(g, t3):
          o = pl.multiple_of(g * GB, GB)
          pltpu.make_async_copy(prow_hbm.at[rix, wid2, pl.ds(o, GB)],
                                idxb.at[t3], sem_st.at[t3]).wait()
          pltpu.make_async_copy(pcol_hbm.at[rix, wid2, pl.ds(o, GB)],
                                sidx.at[t3], sem_st.at[t3]).wait()
          pltpu.make_async_copy(pew_hbm.at[rix, wid2, pl.ds(o, GB)],
                                ewst.at[t3], sem_st.at[t3]).wait()

        def gissue(t2, t3):
          pltpu.async_copy(u_hbm.at[idxb.at[t3]],
                           gbuf.at[pl.ds(t2 * GB, GB), :], sem_g.at[t2])

        def gwait(t2, t3):
          pltpu.make_async_copy(u_hbm.at[idxb.at[t3]],
                                gbuf.at[pl.ds(t2 * GB, GB), :],
                                sem_g.at[t2]).wait()

        stage(0, 0)
        stage_wait(0, 0)
        gissue(0, 0)

        @pl.when(nb > 1)
        def _():
          stage(1, 1)

        def blk(g, _):
          t3 = lax.rem(g, 3)
          t2 = lax.rem(g, 2)
          t3n = lax.rem(g + 1, 3)
          t2n = 1 - t2

          @pl.when(g + 2 < nb)
          def _():
            stage(g + 2, lax.rem(g + 2, 3))

          @pl.when(g + 1 < nb)
          def _():
            stage_wait(g + 1, t3n)
            gissue(t2n, t3n)

          gwait(t2, t3)

          rbase = t2 * GB

          def mul(q, _):
            ewv = ewst[t3, pl.ds(q * LANES, LANES)]
            for e in range(LANES):
              sc = ewv.at[jnp.full((LANES,), e, jnp.int32)].get(
                  mode="promise_in_bounds")
              rr = rbase + q * LANES + e
              for fb in range(h // LANES):
                cur = gbuf[rr, pl.ds(fb * LANES, LANES)]
                gbuf[rr, pl.ds(fb * LANES, LANES)] = cur * sc
            return 0
          lax.fori_loop(0, GB // LANES, mul, 0)

          pltpu.sync_copy(gbuf.at[pl.ds(rbase, GB), :], acc.at[sidx.at[t3]],
                          add=True)
          return 0
        lax.fori_loop(0, nb, blk, 0)

      plsc.subcore_barrier()
      for z in range(tr // cz):
        stripe = pl.multiple_of(s * tr + z * cz, cz)
        dsto = pl.multiple_of(lo + s * tr + z * cz, cz)
        pltpu.sync_copy(acc.at[pl.ds(stripe, cz), :],
                        out_hbm.at[pl.ds(dsto, cz), :])
      plsc.subcore_barrier()

  return pl.kernel(
      body,
      out_type=jax.ShapeDtypeStruct((npad, h), jnp.float32),
      mesh=_mesh(),
      scratch_types=[
          pltpu.VMEM((3, GB), jnp.int32),
          pltpu.VMEM((3, GB), jnp.int32),
          pltpu.VMEM((3, GB), jnp.float32),
          pltpu.VMEM((2 * GB, h), jnp.float32),
          pltpu.VMEM((cz, h), jnp.float32),
          pltpu.VMEM((NW, LANES), jnp.int32),
          pltpu.VMEM_SHARED((rng, h), jnp.float32),
          pltpu.SemaphoreType.DMA((3,)),
          pltpu.SemaphoreType.DMA((2,)),
      ],
      compiler_params=_SC_PARAMS,
  )


def _tc_first(x, w1, degsum, bn):
  n, din = x.shape
  h = w1.shape[1]
  grid = (n // bn,)

  def body(xb, wb, db, xw_o, u_o, dis_o):
    d = lax.rsqrt(db[...])
    xw = jnp.dot(xb[...], wb[...], preferred_element_type=jnp.float32)
    xw_o[...] = xw
    u_o[...] = xw * d
    dis_o[...] = d

  return pl.pallas_call(
      body,
      grid=grid,
      in_specs=[
          pl.BlockSpec((bn, din), lambda i: (i, 0)),
          pl.BlockSpec((din, h), lambda i: (0, 0)),
          pl.BlockSpec((bn, 1), lambda i: (i, 0)),
      ],
      out_specs=[
          pl.BlockSpec((bn, h), lambda i: (i, 0)),
          pl.BlockSpec((bn, h), lambda i: (i, 0)),
          pl.BlockSpec((bn, 1), lambda i: (i, 0)),
      ],
      out_shape=[
          jax.ShapeDtypeStruct((n, h), jnp.float32),
          jax.ShapeDtypeStruct((n, h), jnp.float32),
          jax.ShapeDtypeStruct((n, 1), jnp.float32),
      ],
  )(x, w1, degsum)


def _tc_mid(s_in, xw, dis, b, wn, bn):
  n, h = xw.shape
  grid = (n // bn,)

  def body(sb, xwb, db, bb, wb, xwn_o, un_o):
    d = db[...]
    pre = d * sb[...] + (d * d) * xwb[...] + bb[...]
    hact = jax.nn.sigmoid(pre)
    xwn = jnp.dot(hact, wb[...], preferred_element_type=jnp.float32)
    xwn_o[...] = xwn
    un_o[...] = xwn * d

  return pl.pallas_call(
      body,
      grid=grid,
      in_specs=[
          pl.BlockSpec((bn, h), lambda i: (i, 0)),
          pl.BlockSpec((bn, h), lambda i: (i, 0)),
          pl.BlockSpec((bn, 1), lambda i: (i, 0)),
          pl.BlockSpec((1, h), lambda i: (0, 0)),
          pl.BlockSpec((h, h), lambda i: (0, 0)),
      ],
      out_specs=[
          pl.BlockSpec((bn, h), lambda i: (i, 0)),
          pl.BlockSpec((bn, h), lambda i: (i, 0)),
      ],
      out_shape=[
          jax.ShapeDtypeStruct((n, h), jnp.float32),
          jax.ShapeDtypeStruct((n, h), jnp.float32),
      ],
  )(s_in, xw, dis, b, wn)


def _tc_final(s_in, xw, dis, b, wl, bl):
  m, h = xw.shape
  out = wl.shape[1]

  def body(sb, xwb, db, bb, wb, blb, o):
    d = db[...]
    pre = d * sb[...] + (d * d) * xwb[...] + bb[...]
    hact = jax.nn.sigmoid(pre)
    o[...] = jnp.dot(hact, wb[...],
                     preferred_element_type=jnp.float32) + blb[...]

  return pl.pallas_call(
      body,
      out_shape=jax.ShapeDtypeStruct((m, out), jnp.float32),
  )(s_in, xw, dis, b, wl, bl)


def kernel(vertex_features, edges, weights, W1, b1, W2, b2, W3, b3, W4, b4,
           Wl, bl):
  n, din = vertex_features.shape
  e = edges.shape[1]
  h = W1.shape[1]

  tr, cz = _pick_stripe(n)
  npad = NRANGES * NSUB * tr
  rng = npad // NRANGES

  ep = _cdiv(e, NW * WP) * NW * WP
  capb = ep // NW + GB
  pad = ep - e
  row = edges[0]
  col = edges[1]
  ew = weights
  if pad:
    pidx = lax.rem(lax.iota(jnp.int32, pad), jnp.int32(n))
    row = jnp.concatenate([row, pidx])
    col = jnp.concatenate([col, pidx])
    ew = jnp.concatenate([ew, jnp.zeros((pad,), jnp.float32)])

  prow, pcol, pew, cnts, deg_part = _part_kernel(npad, ep, rng, capb)(
      row, col, ew)
  degsum = (deg_part[0] + deg_part[1] + 1.0)[:n].reshape(n, 1)

  bn = 2000 if n % 2000 == 0 else 8 * (n // 8)
  while n % bn:
    bn -= 8

  xw, u, dis = _tc_first(vertex_features, W1, degsum, bn)

  spmm = _spmm_kernel(npad, rng, tr, cz, h, capb)
  for wnext, bcur in ((W2, b1), (W3, b2), (W4, b3)):
    s_out = spmm(prow, pcol, pew, cnts, u)
    xw, u = _tc_mid(s_out[:n], xw, dis, bcur.reshape(1, h), wnext, bn)

  s_out = spmm(prow, pcol, pew, cnts, u)
  head = 32
  emb = _tc_final(s_out[:head], xw[:head], dis[:head],
                  b4.reshape(1, h), Wl, bl.reshape(1, Wl.shape[1]))
  return emb[:19]


# speedup vs baseline: 12.5686x; 1.1332x over previous
import jax
import jax.numpy as jnp
from jax import lax
from jax.experimental import pallas as pl
from jax.experimental.pallas import tpu as pltpu
from jax.experimental.pallas import tpu_sc as plsc

NCORES = 2
NSUB = 16
LANES = 16
KPASS = 3
NRANGES = NCORES * KPASS
NW = NCORES * NSUB
GB = 128
WP = 2048

_SC_PARAMS = pltpu.CompilerParams(use_tc_tiling_on_sc=False,
                                  needs_layout_passes=False)


def _cdiv(a, b):
  return -(-a // b)


def _pick_stripe(n):
  tr = _cdiv(_cdiv(n, NRANGES * NSUB), 8) * 8
  while True:
    czs = [d for d in range(64, 257, 8) if tr % d == 0]
    if czs:
      return tr, max(czs)
    tr += 8


def _mesh():
  return plsc.VectorSubcoreMesh(
      core_axis_name="c", subcore_axis_name="s",
      num_cores=NCORES, num_subcores=NSUB)


def _part_kernel(npad, ep, rng, capb):
  capw = ep // NW
  nwp = capw // WP
  npt = npad // NSUB
  pb = WP + GB

  def body(row_hbm, col_hbm, ew_hbm,
           prow_o, pcol_o, pew_o, cnt_o, deg_o,
           rowb, colb, ewb, col2db, pbrow, pbcol, pbew, zb, cbuf, acc,
           sem_f, sem_d):
    c = lax.axis_index("c")
    s = lax.axis_index("s")
    wid = c * NSUB + s
    base = wid * capw
    iota16 = lax.iota(jnp.int32, LANES)
    zeros16f = jnp.zeros((LANES,), jnp.float32)

    def zb_zero(i, _):
      zb[pl.ds(i * LANES, LANES)] = zeros16f
      return 0
    lax.fori_loop(0, npt // LANES, zb_zero, 0)
    pltpu.sync_copy(zb, acc.at[pl.ds(pl.multiple_of(s * npt, npt), npt)])
    plsc.subcore_barrier()

    def window(wi, carry):
      off = pl.multiple_of(base + wi * WP, WP)
      pltpu.sync_copy(row_hbm.at[pl.ds(off, WP)], rowb)
      pltpu.sync_copy(col_hbm.at[pl.ds(off, WP)], colb)
      pltpu.sync_copy(ew_hbm.at[pl.ds(off, WP)], ewb)

      def dchunk(j, _):
        for k in range(128 // LANES):
          col2db[j, pl.ds(k * LANES, LANES)] = (
              colb[pl.ds(j * 128 + k * LANES, LANES)])
        pltpu.async_copy(ewb.at[pl.ds(j * 128, 128)], acc.at[col2db.at[j]],
                         sem_d, add=True)
        return 0
      lax.fori_loop(0, WP // 128, dchunk, 0)

      tails = []
      offbs = []
      nfs = []
      for r in range(NRANGES):
        lo = r * rng

        def scan(i, t):
          cols = colb[pl.ds(i * LANES, LANES)]
          msk = (cols >= lo) & (cols < lo + rng)
          mi = msk.astype(jnp.int32)
          pos = t + plsc.cumsum(mi) - 1
          plsc.store_scatter(pbrow.at[r], [pos],
                             rowb[pl.ds(i * LANES, LANES)], mask=msk)
          plsc.store_scatter(pbcol.at[r], [pos], cols - lo, mask=msk)
          plsc.store_scatter(pbew.at[r], [pos],
                             ewb[pl.ds(i * LANES, LANES)], mask=msk)
          return t + jnp.sum(mi)
        t2 = lax.fori_loop(0, WP // LANES, scan, carry[r])
        nf = t2 // GB
        offb = carry[NRANGES + r]

        def flush(b, _):
          src = pl.multiple_of(b * GB, GB)
          dst = pl.multiple_of((offb + b) * GB, GB)
          pltpu.async_copy(pbrow.at[r, pl.ds(src, GB)],
                           prow_o.at[r, wid, pl.ds(dst, GB)], sem_f)
          pltpu.async_copy(pbcol.at[r, pl.ds(src, GB)],
                           pcol_o.at[r, wid, pl.ds(dst, GB)], sem_f)
          pltpu.async_copy(pbew.at[r, pl.ds(src, GB)],
                           pew_o.at[r, wid, pl.ds(dst, GB)], sem_f)
          return 0
        lax.fori_loop(0, nf, flush, 0)
        tails.append(t2 - nf * GB)
        offbs.append(offb + nf)
        nfs.append(nf)

      for r in range(NRANGES):
        offb = carry[NRANGES + r]

        def fwait(b, _):
          src = pl.multiple_of(b * GB, GB)
          dst = pl.multiple_of((offb + b) * GB, GB)
          pltpu.make_async_copy(pbrow.at[r, pl.ds(src, GB)],
                                prow_o.at[r, wid, pl.ds(dst, GB)],
                                sem_f).wait()
          pltpu.make_async_copy(pbcol.at[r, pl.ds(src, GB)],
                                pcol_o.at[r, wid, pl.ds(dst, GB)],
                                sem_f).wait()
          pltpu.make_async_copy(pbew.at[r, pl.ds(src, GB)],
                                pew_o.at[r, wid, pl.ds(dst, GB)],
                                sem_f).wait()
          return 0
        lax.fori_loop(0, nfs[r], fwait, 0)

      pltpu.make_async_copy(ew_hbm.at[pl.ds(off, WP)], ewb, sem_d).wait()

      for r in range(NRANGES):
        srcoff = pl.multiple_of(nfs[r] * GB, GB)
        for k in range(GB // LANES):
          vr = pbrow[r, pl.ds(srcoff + k * LANES, LANES)]
          vc = pbcol[r, pl.ds(srcoff + k * LANES, LANES)]
          ve = pbew[r, pl.ds(srcoff + k * LANES, LANES)]
          pbrow[r, pl.ds(k * LANES, LANES)] = vr
          pbcol[r, pl.ds(k * LANES, LANES)] = vc
          pbew[r, pl.ds(k * LANES, LANES)] = ve

      return tuple(tails) + tuple(offbs)

    carry = lax.fori_loop(0, nwp, window,
                          tuple(jnp.int32(0) for _ in range(2 * NRANGES)))

    cv = jnp.zeros((LANES,), jnp.int32)
    for r in range(NRANGES):
      tail = carry[r]
      offb = carry[NRANGES + r]
      for i in range(GB // LANES):
        pos = tail + i * LANES + iota16
        plsc.store_scatter(pbrow.at[r], [pos], iota16)
        plsc.store_scatter(pbcol.at[r], [pos], iota16)
        plsc.store_scatter(pbew.at[r], [pos], zeros16f)
      dst = pl.multiple_of(offb * GB, GB)
      pltpu.async_copy(pbrow.at[r, pl.ds(0, GB)],
                       prow_o.at[r, wid, pl.ds(dst, GB)], sem_f)
      pltpu.async_copy(pbcol.at[r, pl.ds(0, GB)],
                       pcol_o.at[r, wid, pl.ds(dst, GB)], sem_f)
      pltpu.async_copy(pbew.at[r, pl.ds(0, GB)],
                       pew_o.at[r, wid, pl.ds(dst, GB)], sem_f)
      cv = cv + jnp.where(iota16 == r, offb + 1, 0)
    for r in range(NRANGES):
      offb = carry[NRANGES + r]
      dst = pl.multiple_of(offb * GB, GB)
      pltpu.make_async_copy(pbrow.at[r, pl.ds(0, GB)],
                            prow_o.at[r, wid, pl.ds(dst, GB)], sem_f).wait()
      pltpu.make_async_copy(pbcol.at[r, pl.ds(0, GB)],
                            pcol_o.at[r, wid, pl.ds(dst, GB)], sem_f).wait()
      pltpu.make_async_copy(pbew.at[r, pl.ds(0, GB)],
                            pew_o.at[r, wid, pl.ds(dst, GB)], sem_f).wait()

    cbuf[pl.ds(0, LANES)] = cv
    pltpu.sync_copy(cbuf, cnt_o.at[wid])

    plsc.subcore_barrier()
    pltpu.sync_copy(acc.at[pl.ds(pl.multiple_of(s * npt, npt), npt)],
                    deg_o.at[c, pl.ds(pl.multiple_of(s * npt, npt), npt)])

  return pl.kernel(
      body,
      out_type=[
          jax.ShapeDtypeStruct((NRANGES, NW, capb), jnp.int32),
          jax.ShapeDtypeStruct((NRANGES, NW, capb), jnp.int32),
          jax.ShapeDtypeStruct((NRANGES, NW, capb), jnp.float32),
          jax.ShapeDtypeStruct((NW, LANES), jnp.int32),
          jax.ShapeDtypeStruct((NCORES, npad), jnp.float32),
      ],
      mesh=_mesh(),
      scratch_types=[
          pltpu.VMEM((WP,), jnp.int32),
          pltpu.VMEM((WP,), jnp.int32),
          pltpu.VMEM((WP,), jnp.float32),
          pltpu.VMEM((WP // 128, 128), jnp.int32),
          pltpu.VMEM((NRANGES, pb), jnp.int32),
          pltpu.VMEM((NRANGES, pb), jnp.int32),
          pltpu.VMEM((NRANGES, pb), jnp.float32),
          pltpu.VMEM((npad // NSUB,), jnp.float32),
          pltpu.VMEM((LANES,), jnp.int32),
          pltpu.VMEM_SHARED((npad,), jnp.float32),
          pltpu.SemaphoreType.DMA,
          pltpu.SemaphoreType.DMA,
      ],
      compiler_params=_SC_PARAMS,
  )


def _spmm_kernel(npad, rng, tr, cz, h, capb):
  def body(prow_hbm, pcol_hbm, pew_hbm, cnt_hbm, u_hbm, out_hbm,
           idxb, sidx, ewst, gbuf, zb, cntb, acc, sem_st, sem_g, sem_sc):
    c = lax.axis_index("c")
    s = lax.axis_index("s")

    def zb_zero(i, _):
      for fb in range(h // LANES):
        zb[i, pl.ds(fb * LANES, LANES)] = jnp.zeros((LANES,), jnp.float32)
      return 0
    lax.fori_loop(0, cz, zb_zero, 0)

    pltpu.sync_copy(cnt_hbm, cntb)

    for p in range(KPASS):
      rix = c * KPASS + p
      lo = rix * rng

      for z in range(tr // cz):
        pltpu.sync_copy(
            zb, acc.at[pl.ds(pl.multiple_of(s * tr + z * cz, cz), cz), :])
      plsc.subcore_barrier()

      for wsub in range(2):
        wid2 = s * 2 + wsub
        cvec = cntb[wid2, pl.ds(0, LANES)]
        nbv = cvec.at[jnp.full((LANES,), rix, jnp.int32)].get(
            mode="promise_in_bounds")
        nb = jnp.squeeze(lax.slice(nbv, (0,), (1,)))

        def stage(g, t4):
          o = pl.multiple_of(g * GB, GB)
          pltpu.async_copy(prow_hbm.at[rix, wid2, pl.ds(o, GB)],
                           idxb.at[t4], sem_st.at[t4])
          pltpu.async_copy(pcol_hbm.at[rix, wid2, pl.ds(o, GB)],
                           sidx.at[t4], sem_st.at[t4])
          pltpu.async_copy(pew_hbm.at[rix, wid2, pl.ds(o, GB)],
                           ewst.at[t4], sem_st.at[t4])

        def stage_wait(g, t4):
          o = pl.multiple_of(g * GB, GB)
          pltpu.make_async_copy(prow_hbm.at[rix, wid2, pl.ds(o, GB)],
                                idxb.at[t4], sem_st.at[t4]).wait()
          pltpu.make_async_copy(pcol_hbm.at[rix, wid2, pl.ds(o, GB)],
                                sidx.at[t4], sem_st.at[t4]).wait()
          pltpu.make_async_copy(pew_hbm.at[rix, wid2, pl.ds(o, GB)],
                                ewst.at[t4], sem_st.at[t4]).wait()

        def gissue(t4):
          pltpu.async_copy(u_hbm.at[idxb.at[t4]],
                           gbuf.at[pl.ds(t4 * GB, GB), :], sem_g.at[t4])

        def gwait(t4):
          pltpu.make_async_copy(u_hbm.at[idxb.at[t4]],
                                gbuf.at[pl.ds(t4 * GB, GB), :],
                                sem_g.at[t4]).wait()

        def sc_drain(t4):
          pltpu.make_async_copy(u_hbm.at[pl.ds(0, GB), :],
                                gbuf.at[pl.ds(t4 * GB, GB), :],
                                sem_sc.at[t4]).wait()

        stage(0, 0)

        @pl.when(nb > 1)
        def _():
          stage(1, 1)
        stage_wait(0, 0)
        gissue(0)

        def blk(g, _):
          t4 = lax.rem(g, 4)
          t4n = lax.rem(g + 1, 4)

          @pl.when(g >= 2)
          def _():
            sc_drain(lax.rem(g + 2, 4))

          @pl.when(g + 2 < nb)
          def _():
            stage(g + 2, lax.rem(g + 2, 4))

          @pl.when(g + 1 < nb)
          def _():
            stage_wait(g + 1, t4n)
            gissue(t4n)

          gwait(t4)

          rbase = t4 * GB

          def mul(q, _):
            ewv = ewst[t4, pl.ds(q * LANES, LANES)]
            for e in range(LANES):
              sc = ewv.at[jnp.full((LANES,), e, jnp.int32)].get(
                  mode="promise_in_bounds")
              rr = rbase + q * LANES + e
              for fb in range(h // LANES):
                cur = gbuf[rr, pl.ds(fb * LANES, LANES)]
                gbuf[rr, pl.ds(fb * LANES, LANES)] = cur * sc
            return 0
          lax.fori_loop(0, GB // LANES, mul, 0)

          pltpu.async_copy(gbuf.at[pl.ds(rbase, GB), :], acc.at[sidx.at[t4]],
                           sem_sc.at[t4], add=True)
          return 0
        lax.fori_loop(0, nb, blk, 0)

        @pl.when(nb >= 2)
        def _():
          sc_drain(lax.rem(nb - 2, 4))
        sc_drain(lax.rem(nb - 1, 4))

      plsc.subcore_barrier()
      for z in range(tr // cz):
        stripe = pl.multiple_of(s * tr + z * cz, cz)
        dsto = pl.multiple_of(lo + s * tr + z * cz, cz)
        pltpu.sync_copy(acc.at[pl.ds(stripe, cz), :],
                        out_hbm.at[pl.ds(dsto, cz), :])
      plsc.subcore_barrier()

  return pl.kernel(
      body,
      out_type=jax.ShapeDtypeStruct((npad, h), jnp.float32),
      mesh=_mesh(),
      scratch_types=[
          pltpu.VMEM((4, GB), jnp.int32),
          pltpu.VMEM((4, GB), jnp.int32),
          pltpu.VMEM((4, GB), jnp.float32),
          pltpu.VMEM((4 * GB, h), jnp.float32),
          pltpu.VMEM((cz, h), jnp.float32),
          pltpu.VMEM((NW, LANES), jnp.int32),
          pltpu.VMEM_SHARED((rng, h), jnp.float32),
          pltpu.SemaphoreType.DMA((4,)),
          pltpu.SemaphoreType.DMA((4,)),
          pltpu.SemaphoreType.DMA((4,)),
      ],
      compiler_params=_SC_PARAMS,
  )


def _tc_first(x, w1, degsum, bn):
  n, din = x.shape
  h = w1.shape[1]
  grid = (n // bn,)

  def body(xb, wb, db, xw_o, u_o, dis_o):
    d = lax.rsqrt(db[...])
    xw = jnp.dot(xb[...], wb[...], preferred_element_type=jnp.float32)
    xw_o[...] = xw
    u_o[...] = xw * d
    dis_o[...] = d

  return pl.pallas_call(
      body,
      grid=grid,
      in_specs=[
          pl.BlockSpec((bn, din), lambda i: (i, 0)),
          pl.BlockSpec((din, h), lambda i: (0, 0)),
          pl.BlockSpec((bn, 1), lambda i: (i, 0)),
      ],
      out_specs=[
          pl.BlockSpec((bn, h), lambda i: (i, 0)),
          pl.BlockSpec((bn, h), lambda i: (i, 0)),
          pl.BlockSpec((bn, 1), lambda i: (i, 0)),
      ],
      out_shape=[
          jax.ShapeDtypeStruct((n, h), jnp.float32),
          jax.ShapeDtypeStruct((n, h), jnp.float32),
          jax.ShapeDtypeStruct((n, 1), jnp.float32),
      ],
  )(x, w1, degsum)


def _tc_mid(s_in, xw, dis, b, wn, bn):
  n, h = xw.shape
  grid = (n // bn,)

  def body(sb, xwb, db, bb, wb, xwn_o, un_o):
    d = db[...]
    pre = d * sb[...] + (d * d) * xwb[...] + bb[...]
    hact = jax.nn.sigmoid(pre)
    xwn = jnp.dot(hact, wb[...], preferred_element_type=jnp.float32)
    xwn_o[...] = xwn
    un_o[...] = xwn * d

  return pl.pallas_call(
      body,
      grid=grid,
      in_specs=[
          pl.BlockSpec((bn, h), lambda i: (i, 0)),
          pl.BlockSpec((bn, h), lambda i: (i, 0)),
          pl.BlockSpec((bn, 1), lambda i: (i, 0)),
          pl.BlockSpec((1, h), lambda i: (0, 0)),
          pl.BlockSpec((h, h), lambda i: (0, 0)),
      ],
      out_specs=[
          pl.BlockSpec((bn, h), lambda i: (i, 0)),
          pl.BlockSpec((bn, h), lambda i: (i, 0)),
      ],
      out_shape=[
          jax.ShapeDtypeStruct((n, h), jnp.float32),
          jax.ShapeDtypeStruct((n, h), jnp.float32),
      ],
  )(s_in, xw, dis, b, wn)


def _tc_final(s_in, xw, dis, b, wl, bl):
  m, h = xw.shape
  out = wl.shape[1]

  def body(sb, xwb, db, bb, wb, blb, o):
    d = db[...]
    pre = d * sb[...] + (d * d) * xwb[...] + bb[...]
    hact = jax.nn.sigmoid(pre)
    o[...] = jnp.dot(hact, wb[...],
                     preferred_element_type=jnp.float32) + blb[...]

  return pl.pallas_call(
      body,
      out_shape=jax.ShapeDtypeStruct((m, out), jnp.float32),
  )(s_in, xw, dis, b, wl, bl)


def kernel(vertex_features, edges, weights, W1, b1, W2, b2, W3, b3, W4, b4,
           Wl, bl):
  n, din = vertex_features.shape
  e = edges.shape[1]
  h = W1.shape[1]

  tr, cz = _pick_stripe(n)
  npad = NRANGES * NSUB * tr
  rng = npad // NRANGES

  ep = _cdiv(e, NW * WP) * NW * WP
  capb = ep // NW + GB
  pad = ep - e
  row = edges[0]
  col = edges[1]
  ew = weights
  if pad:
    pidx = lax.rem(lax.iota(jnp.int32, pad), jnp.int32(n))
    row = jnp.concatenate([row, pidx])
    col = jnp.concatenate([col, pidx])
    ew = jnp.concatenate([ew, jnp.zeros((pad,), jnp.float32)])

  prow, pcol, pew, cnts, deg_part = _part_kernel(npad, ep, rng, capb)(
      row, col, ew)
  degsum = (deg_part[0] + deg_part[1] + 1.0)[:n].reshape(n, 1)

  bn = 2000 if n % 2000 == 0 else 8 * (n // 8)
  while n % bn:
    bn -= 8

  xw, u, dis = _tc_first(vertex_features, W1, degsum, bn)

  spmm = _spmm_kernel(npad, rng, tr, cz, h, capb)
  for wnext, bcur in ((W2, b1), (W3, b2), (W4, b3)):
    s_out = spmm(prow, pcol, pew, cnts, u)
    xw, u = _tc_mid(s_out[:n], xw, dis, bcur.reshape(1, h), wnext, bn)

  s_out = spmm(prow, pcol, pew, cnts, u)
  head = 32
  emb = _tc_final(s_out[:head], xw[:head], dis[:head],
                  b4.reshape(1, h), Wl, bl.reshape(1, Wl.shape[1]))
  return emb[:19]
